# baseline (speedup 1.0000x reference)
import jax
import jax.numpy as jnp
import numpy as np
from jax.experimental import pallas as pl
from jax.experimental.pallas import tpu as pltpu

B = 4
N = 4096
IN_FEATURES = 3
NUM_CLASSES = 16
NUM_PARTS = 16
SA_CFG = (
    (1024, (0.1, 0.2), (16, 32), ((16, 16, 32), (32, 32, 64))),
    (256, (0.2, 0.4), (16, 32), ((64, 64, 128), (64, 96, 128))),
    (64, (0.4, 0.8), (16, 32), ((128, 196, 256), (128, 196, 256))),
    (16, (0.8, 1.6), (16, 32), ((256, 256, 512), (256, 384, 512))),
)
FP_DIMS = ((512, 512), (512, 512), (256, 256), (128, 128))


def _conv(x, L):
    y = jnp.einsum('oc,bc...->bo...', L["W"], x)
    return y + L["b"].reshape((1, -1) + (1,) * (y.ndim - 2))


def _bn(x, L):
    axes = (0,) + tuple(range(2, x.ndim))
    mean = jnp.mean(x, axis=axes, keepdims=True)
    var = jnp.var(x, axis=axes, keepdims=True)
    sh = (1, -1) + (1,) * (x.ndim - 2)
    return (x - mean) / jnp.sqrt(var + 1e-5) * L["g"].reshape(sh) + L["be"].reshape(sh)


def _mlp(x, layers):
    for L in layers:
        x = jax.nn.relu(_bn(_conv(x, L), L))
    return x


def _fps(xyz, m):
    b, n, _ = xyz.shape

    def body(carry, _):
        dist, last = carry
        last_xyz = jnp.take_along_axis(xyz, last[:, None, None], axis=1)
        d = jnp.sum((xyz - last_xyz) ** 2, axis=-1)
        dist = jnp.minimum(dist, d)
        nxt = jnp.argmax(dist, axis=-1).astype(jnp.int32)
        return (dist, nxt), last

    init = (jnp.full((b, n), 1e10, jnp.float32), jnp.zeros((b,), jnp.int32))
    _, idxs = jax.lax.scan(body, init, None, length=m)
    return jnp.transpose(idxs)


def _gather_pts(p, idx):
    return jax.vmap(lambda pp, ii: pp[ii])(p, idx)


def _gather_feats(f, idx):
    return jax.vmap(lambda ff, ii: ff[:, ii])(f, idx)


def _ball_query(xyz, new_xyz, radius, nsample):
    n = xyz.shape[1]
    d2 = jnp.sum((new_xyz[:, :, None, :] - xyz[:, None, :, :]) ** 2, axis=-1)
    mask = d2 <= radius * radius
    keys = jnp.where(mask, jnp.arange(n)[None, None, :], n)
    order = jnp.argsort(keys, axis=-1)[:, :, :nsample]
    cnt = jnp.sum(mask, axis=-1, keepdims=True)
    idx = jnp.where(jnp.arange(nsample)[None, None, :] < cnt, order, order[:, :, :1])
    return idx


def _set_abstraction(xyz, feats, m, radii, nsamples, scales):
    fidx = _fps(xyz, m)
    new_xyz = _gather_pts(xyz, fidx)
    outs = []
    for r, s, mlp in zip(radii, nsamples, scales):
        idx = _ball_query(xyz, new_xyz, r, s)
        gx = _gather_pts(xyz, idx) - new_xyz[:, :, None, :]
        h = jnp.transpose(gx, (0, 3, 1, 2))
        if feats is not None:
            h = jnp.concatenate([h, _gather_feats(feats, idx)], axis=1)
        h = _mlp(h, mlp)
        outs.append(jnp.max(h, axis=-1))
    return new_xyz, jnp.concatenate(outs, axis=1)


def _feature_prop(ux, kx, uf, kf, mlp):
    d2 = jnp.sum((ux[:, :, None, :] - kx[:, None, :, :]) ** 2, axis=-1)
    negd, idx = jax.lax.top_k(-d2, 3)
    w = 1.0 / (jnp.maximum(-negd, 0.0) + 1e-8)
    w = w / jnp.sum(w, axis=-1, keepdims=True)
    interp = jnp.sum(_gather_feats(kf, idx) * w[:, None, :, :], axis=-1)
    h = jnp.concatenate([interp, uf], axis=1) if uf is not None else interp
    return _mlp(h, mlp)


def _identity_kernel(x_ref, o_ref):
    o_ref[...] = x_ref[...]


def _pallas_identity(x):
    return pl.pallas_call(
        _identity_kernel,
        out_shape=jax.ShapeDtypeStruct(x.shape, x.dtype),
    )(x)


def kernel(points, params):
    bb, nn = points.shape[0], points.shape[1]
    xyz = points[..., :3]
    feats = jnp.transpose(points[..., 3:], (0, 2, 1)) if points.shape[-1] > 3 else None
    xyz_list, feat_list = [xyz], [feats]
    cx, cf = xyz, feats
    for (m, radii, ns, _), scales in zip(SA_CFG, params["sa"]):
        cx, cf = _set_abstraction(cx, cf, m, radii, ns, scales)
        xyz_list.append(cx)
        feat_list.append(cf)
    t = -2
    for mlp in params["fp"]:
        feat_list[t] = _feature_prop(xyz_list[t], xyz_list[t + 1], feat_list[t], feat_list[t + 1], mlp)
        t -= 1
    f0 = feat_list[0]
    fl1, fl2 = params["final"]
    h = jax.nn.relu(_bn(_conv(f0, fl1), fl1))
    parts = _conv(h, fl2)
    parts_sm = jax.nn.softmax(parts, axis=1)
    pp1, ppg = params["part"]
    h2 = jax.nn.relu(_conv(f0, pp1))
    h2 = h2.reshape(bb, NUM_PARTS, 128, nn)
    pred = jnp.einsum('pgc,bpcn->bpgn', ppg["W"], h2) + ppg["b"][None, :, :, None]
    pred = pred.reshape(bb, NUM_PARTS * NUM_CLASSES, nn)
    weighted = (pred.reshape(bb, NUM_CLASSES, NUM_PARTS, nn) * parts_sm[:, None, :, :]).sum(axis=2)
    out = jnp.concatenate([parts, weighted], axis=1)
    return _pallas_identity(out)


# speedup vs baseline: 1.3335x; 1.3335x over previous
import jax
import jax.numpy as jnp
import numpy as np
from jax.experimental import pallas as pl
from jax.experimental.pallas import tpu as pltpu

B = 4
N = 4096
IN_FEATURES = 3
NUM_CLASSES = 16
NUM_PARTS = 16
SA_CFG = (
    (1024, (0.1, 0.2), (16, 32), ((16, 16, 32), (32, 32, 64))),
    (256, (0.2, 0.4), (16, 32), ((64, 64, 128), (64, 96, 128))),
    (64, (0.4, 0.8), (16, 32), ((128, 196, 256), (128, 196, 256))),
    (16, (0.8, 1.6), (16, 32), ((256, 256, 512), (256, 384, 512))),
)
FP_DIMS = ((512, 512), (512, 512), (256, 256), (128, 128))


def _conv(x, L):
    y = jnp.einsum('oc,bc...->bo...', L["W"], x)
    return y + L["b"].reshape((1, -1) + (1,) * (y.ndim - 2))


def _bn(x, L):
    axes = (0,) + tuple(range(2, x.ndim))
    mean = jnp.mean(x, axis=axes, keepdims=True)
    var = jnp.var(x, axis=axes, keepdims=True)
    sh = (1, -1) + (1,) * (x.ndim - 2)
    return (x - mean) / jnp.sqrt(var + 1e-5) * L["g"].reshape(sh) + L["be"].reshape(sh)


def _mlp(x, layers):
    for L in layers:
        x = jax.nn.relu(_bn(_conv(x, L), L))
    return x


def _fps_body(m, n, xT_ref, nx_ref):
    xT = xT_ref[0]
    lane_n = jax.lax.broadcasted_iota(jnp.int32, (1, n), 1)
    lane_m = jax.lax.broadcasted_iota(jnp.int32, (1, m), 1)

    def step(i, carry):
        dist, last, nx = carry
        lx = jnp.sum(jnp.where(lane_n == last, xT, 0.0), axis=1, keepdims=True)
        nx = jnp.where(lane_m == i, lx, nx)
        d = jnp.sum((xT - lx) ** 2, axis=0, keepdims=True)
        dist = jnp.minimum(dist, d)
        nxt = jnp.argmax(dist, axis=1, keepdims=True).astype(jnp.int32)
        return dist, nxt, nx

    init = (jnp.full((1, n), 1e10, jnp.float32), jnp.zeros((1, 1), jnp.int32),
            jnp.zeros((3, m), jnp.float32))
    _, _, nx = jax.lax.fori_loop(0, m, step, init)
    nx_ref[0] = nx


def _fps_new_xyz(xyz, m, interpret=False):
    b, n, _ = xyz.shape
    xT = jnp.transpose(xyz, (0, 2, 1))
    import functools
    nxT = pl.pallas_call(
        functools.partial(_fps_body, m, n),
        grid=(b,),
        in_specs=[pl.BlockSpec((1, 3, n), lambda i: (i, 0, 0))],
        out_specs=pl.BlockSpec((1, 3, m), lambda i: (i, 0, 0)),
        out_shape=jax.ShapeDtypeStruct((b, 3, m), jnp.float32),
        interpret=interpret,
    )(xT)
    return jnp.transpose(nxT, (0, 2, 1))


def _gather_pts(p, idx):
    return jax.vmap(lambda pp, ii: pp[ii])(p, idx)


def _gather_feats(f, idx):
    return jax.vmap(lambda ff, ii: ff[:, ii])(f, idx)


def _ball_query(xyz, new_xyz, radius, nsample):
    n = xyz.shape[1]
    d2 = jnp.sum((new_xyz[:, :, None, :] - xyz[:, None, :, :]) ** 2, axis=-1)
    mask = d2 <= radius * radius
    keys = jnp.where(mask, jnp.arange(n)[None, None, :], n)
    order = jnp.argsort(keys, axis=-1)[:, :, :nsample]
    cnt = jnp.sum(mask, axis=-1, keepdims=True)
    idx = jnp.where(jnp.arange(nsample)[None, None, :] < cnt, order, order[:, :, :1])
    return idx


def _set_abstraction(xyz, feats, m, radii, nsamples, scales):
    new_xyz = _fps_new_xyz(xyz, m)
    outs = []
    for r, s, mlp in zip(radii, nsamples, scales):
        idx = _ball_query(xyz, new_xyz, r, s)
        gx = _gather_pts(xyz, idx) - new_xyz[:, :, None, :]
        h = jnp.transpose(gx, (0, 3, 1, 2))
        if feats is not None:
            h = jnp.concatenate([h, _gather_feats(feats, idx)], axis=1)
        h = _mlp(h, mlp)
        outs.append(jnp.max(h, axis=-1))
    return new_xyz, jnp.concatenate(outs, axis=1)


def _feature_prop(ux, kx, uf, kf, mlp):
    d2 = jnp.sum((ux[:, :, None, :] - kx[:, None, :, :]) ** 2, axis=-1)
    negd, idx = jax.lax.top_k(-d2, 3)
    w = 1.0 / (jnp.maximum(-negd, 0.0) + 1e-8)
    w = w / jnp.sum(w, axis=-1, keepdims=True)
    interp = jnp.sum(_gather_feats(kf, idx) * w[:, None, :, :], axis=-1)
    h = jnp.concatenate([interp, uf], axis=1) if uf is not None else interp
    return _mlp(h, mlp)


def _identity_kernel(x_ref, o_ref):
    o_ref[...] = x_ref[...]


def _pallas_identity(x):
    return pl.pallas_call(
        _identity_kernel,
        out_shape=jax.ShapeDtypeStruct(x.shape, x.dtype),
    )(x)


def kernel(points, params):
    bb, nn = points.shape[0], points.shape[1]
    xyz = points[..., :3]
    feats = jnp.transpose(points[..., 3:], (0, 2, 1)) if points.shape[-1] > 3 else None
    xyz_list, feat_list = [xyz], [feats]
    cx, cf = xyz, feats
    for (m, radii, ns, _), scales in zip(SA_CFG, params["sa"]):
        cx, cf = _set_abstraction(cx, cf, m, radii, ns, scales)
        xyz_list.append(cx)
        feat_list.append(cf)
    t = -2
    for mlp in params["fp"]:
        feat_list[t] = _feature_prop(xyz_list[t], xyz_list[t + 1], feat_list[t], feat_list[t + 1], mlp)
        t -= 1
    f0 = feat_list[0]
    fl1, fl2 = params["final"]
    h = jax.nn.relu(_bn(_conv(f0, fl1), fl1))
    parts = _conv(h, fl2)
    parts_sm = jax.nn.softmax(parts, axis=1)
    pp1, ppg = params["part"]
    h2 = jax.nn.relu(_conv(f0, pp1))
    h2 = h2.reshape(bb, NUM_PARTS, 128, nn)
    pred = jnp.einsum('pgc,bpcn->bpgn', ppg["W"], h2) + ppg["b"][None, :, :, None]
    pred = pred.reshape(bb, NUM_PARTS * NUM_CLASSES, nn)
    weighted = (pred.reshape(bb, NUM_CLASSES, NUM_PARTS, nn) * parts_sm[:, None, :, :]).sum(axis=2)
    out = jnp.concatenate([parts, weighted], axis=1)
    return _pallas_identity(out)


# speedup vs baseline: 2.1745x; 1.6306x over previous
import functools

import jax
import jax.numpy as jnp
import numpy as np
from jax.experimental import pallas as pl
from jax.experimental.pallas import tpu as pltpu

B = 4
N = 4096
IN_FEATURES = 3
NUM_CLASSES = 16
NUM_PARTS = 16
SA_CFG = (
    (1024, (0.1, 0.2), (16, 32), ((16, 16, 32), (32, 32, 64))),
    (256, (0.2, 0.4), (16, 32), ((64, 64, 128), (64, 96, 128))),
    (64, (0.4, 0.8), (16, 32), ((128, 196, 256), (128, 196, 256))),
    (16, (0.8, 1.6), (16, 32), ((256, 256, 512), (256, 384, 512))),
)
FP_DIMS = ((512, 512), (512, 512), (256, 256), (128, 128))


def _conv(x, L):
    y = jnp.einsum('oc,bc...->bo...', L["W"], x)
    return y + L["b"].reshape((1, -1) + (1,) * (y.ndim - 2))


def _bn(x, L):
    axes = (0,) + tuple(range(2, x.ndim))
    mean = jnp.mean(x, axis=axes, keepdims=True)
    var = jnp.var(x, axis=axes, keepdims=True)
    sh = (1, -1) + (1,) * (x.ndim - 2)
    return (x - mean) / jnp.sqrt(var + 1e-5) * L["g"].reshape(sh) + L["be"].reshape(sh)


def _mlp(x, layers):
    for L in layers:
        x = jax.nn.relu(_bn(_conv(x, L), L))
    return x


def _fps_body(m, n, xT_ref, nx_ref):
    xT = xT_ref[0]
    lane_n = jax.lax.broadcasted_iota(jnp.int32, (1, n), 1)
    lane_m = jax.lax.broadcasted_iota(jnp.int32, (1, m), 1)

    def step(i, carry):
        dist, last, nx = carry
        lx = jnp.sum(jnp.where(lane_n == last, xT, 0.0), axis=1, keepdims=True)
        nx = jnp.where(lane_m == i, lx, nx)
        d = jnp.sum((xT - lx) ** 2, axis=0, keepdims=True)
        dist = jnp.minimum(dist, d)
        nxt = jnp.argmax(dist, axis=1, keepdims=True).astype(jnp.int32)
        return dist, nxt, nx

    init = (jnp.full((1, n), 1e10, jnp.float32), jnp.zeros((1, 1), jnp.int32),
            jnp.zeros((3, m), jnp.float32))
    _, _, nx = jax.lax.fori_loop(0, m, step, init)
    nx_ref[0] = nx


def _fps_new_xyz(xyz, m, interpret=False):
    b, n, _ = xyz.shape
    xT = jnp.transpose(xyz, (0, 2, 1))
    import functools
    nxT = pl.pallas_call(
        functools.partial(_fps_body, m, n),
        grid=(b,),
        in_specs=[pl.BlockSpec((1, 3, n), lambda i: (i, 0, 0))],
        out_specs=pl.BlockSpec((1, 3, m), lambda i: (i, 0, 0)),
        out_shape=jax.ShapeDtypeStruct((b, 3, m), jnp.float32),
        interpret=interpret,
    )(xT)
    return jnp.transpose(nxT, (0, 2, 1))


def _gather_pts(p, idx):
    return jax.vmap(lambda pp, ii: pp[ii])(p, idx)


def _gather_feats(f, idx):
    return jax.vmap(lambda ff, ii: ff[:, ii])(f, idx)


def _bq_body(r2, s, n, mb, c_ref, xT_ref, idx_ref):
    c = c_ref[0]
    xT = xT_ref[0]
    d2 = ((c[:, 0:1] - xT[0:1, :]) ** 2 + (c[:, 1:2] - xT[1:2, :]) ** 2) \
        + (c[:, 2:3] - xT[2:3, :]) ** 2
    mask = d2 <= r2
    mi = jnp.where(mask, jnp.int32(1), jnp.int32(0))
    cum = mi
    k = 1
    while k < n:
        shifted = jnp.concatenate(
            [jnp.zeros((mb, k), jnp.int32), cum[:, : n - k]], axis=1)
        cum = cum + shifted
        k *= 2
    cnt = cum[:, n - 1 : n]
    lane_n = jax.lax.broadcasted_iota(jnp.int32, (1, n), 1)
    lane_s = jax.lax.broadcasted_iota(jnp.int32, (1, s), 1)
    buf = jnp.zeros((mb, s), jnp.int32)
    idx0 = jnp.zeros((mb, 1), jnp.int32)
    for t in range(s):
        sel = jnp.logical_and(mask, cum == t + 1)
        it = jnp.sum(jnp.where(sel, lane_n, 0), axis=1, keepdims=True)
        if t == 0:
            idx0 = it
            chosen = it
        else:
            chosen = jnp.where(cnt > t, it, idx0)
        buf = jnp.where(lane_s == t, chosen, buf)
    idx_ref[0] = buf


def _ball_query(xyz, new_xyz, radius, nsample, interpret=False):
    b, n, _ = xyz.shape
    m = new_xyz.shape[1]
    mb = min(m, 256)
    xT = jnp.transpose(xyz, (0, 2, 1))
    idx = pl.pallas_call(
        functools.partial(_bq_body, radius * radius, nsample, n, mb),
        grid=(b, m // mb),
        in_specs=[
            pl.BlockSpec((1, mb, 3), lambda bi, i: (bi, i, 0)),
            pl.BlockSpec((1, 3, n), lambda bi, i: (bi, 0, 0)),
        ],
        out_specs=pl.BlockSpec((1, mb, nsample), lambda bi, i: (bi, i, 0)),
        out_shape=jax.ShapeDtypeStruct((b, m, nsample), jnp.int32),
        interpret=interpret,
    )(new_xyz, xT)
    return idx


def _set_abstraction(xyz, feats, m, radii, nsamples, scales):
    new_xyz = _fps_new_xyz(xyz, m)
    outs = []
    for r, s, mlp in zip(radii, nsamples, scales):
        idx = _ball_query(xyz, new_xyz, r, s)
        gx = _gather_pts(xyz, idx) - new_xyz[:, :, None, :]
        h = jnp.transpose(gx, (0, 3, 1, 2))
        if feats is not None:
            h = jnp.concatenate([h, _gather_feats(feats, idx)], axis=1)
        h = _mlp(h, mlp)
        outs.append(jnp.max(h, axis=-1))
    return new_xyz, jnp.concatenate(outs, axis=1)


def _nn3_body(n, k, ux_ref, kxT_ref, idx_ref, w_ref):
    u = ux_ref[0]
    kT = kxT_ref[0]
    d2 = ((u[:, 0:1] - kT[0:1, :]) ** 2 + (u[:, 1:2] - kT[1:2, :]) ** 2) \
        + (u[:, 2:3] - kT[2:3, :]) ** 2
    lane_k = jax.lax.broadcasted_iota(jnp.int32, (1, k), 1)
    lane_3 = jax.lax.broadcasted_iota(jnp.int32, (1, 3), 1)
    ibuf = jnp.zeros((n, 3), jnp.int32)
    wraw = []
    for t in range(3):
        a = jnp.argmin(d2, axis=1, keepdims=True).astype(jnp.int32)
        v = jnp.min(d2, axis=1, keepdims=True)
        ibuf = jnp.where(lane_3 == t, a, ibuf)
        wraw.append(1.0 / (jnp.maximum(v, 0.0) + 1e-8))
        d2 = jnp.where(lane_k == a, jnp.float32(jnp.inf), d2)
    wsum = (wraw[0] + wraw[1]) + wraw[2]
    wbuf = jnp.where(lane_3 == 0, wraw[0] / wsum,
                     jnp.where(lane_3 == 1, wraw[1] / wsum, wraw[2] / wsum))
    idx_ref[0] = ibuf
    w_ref[0] = wbuf * jnp.ones((n, 3), jnp.float32)


def _nn3(ux, kx, interpret=False):
    b, n, _ = ux.shape
    k = kx.shape[1]
    kxT = jnp.transpose(kx, (0, 2, 1))
    idx, w = pl.pallas_call(
        functools.partial(_nn3_body, n, k),
        grid=(b,),
        in_specs=[
            pl.BlockSpec((1, n, 3), lambda bi: (bi, 0, 0)),
            pl.BlockSpec((1, 3, k), lambda bi: (bi, 0, 0)),
        ],
        out_specs=[
            pl.BlockSpec((1, n, 3), lambda bi: (bi, 0, 0)),
            pl.BlockSpec((1, n, 3), lambda bi: (bi, 0, 0)),
        ],
        out_shape=[
            jax.ShapeDtypeStruct((b, n, 3), jnp.int32),
            jax.ShapeDtypeStruct((b, n, 3), jnp.float32),
        ],
        interpret=interpret,
    )(ux, kxT)
    return idx, w


def _feature_prop(ux, kx, uf, kf, mlp):
    idx, w = _nn3(ux, kx)
    interp = jnp.sum(_gather_feats(kf, idx) * w[:, None, :, :], axis=-1)
    h = jnp.concatenate([interp, uf], axis=1) if uf is not None else interp
    return _mlp(h, mlp)


def _identity_kernel(x_ref, o_ref):
    o_ref[...] = x_ref[...]


def _pallas_identity(x):
    return pl.pallas_call(
        _identity_kernel,
        out_shape=jax.ShapeDtypeStruct(x.shape, x.dtype),
    )(x)


def kernel(points, params):
    bb, nn = points.shape[0], points.shape[1]
    xyz = points[..., :3]
    feats = jnp.transpose(points[..., 3:], (0, 2, 1)) if points.shape[-1] > 3 else None
    xyz_list, feat_list = [xyz], [feats]
    cx, cf = xyz, feats
    for (m, radii, ns, _), scales in zip(SA_CFG, params["sa"]):
        cx, cf = _set_abstraction(cx, cf, m, radii, ns, scales)
        xyz_list.append(cx)
        feat_list.append(cf)
    t = -2
    for mlp in params["fp"]:
        feat_list[t] = _feature_prop(xyz_list[t], xyz_list[t + 1], feat_list[t], feat_list[t + 1], mlp)
        t -= 1
    f0 = feat_list[0]
    fl1, fl2 = params["final"]
    h = jax.nn.relu(_bn(_conv(f0, fl1), fl1))
    parts = _conv(h, fl2)
    parts_sm = jax.nn.softmax(parts, axis=1)
    pp1, ppg = params["part"]
    h2 = jax.nn.relu(_conv(f0, pp1))
    h2 = h2.reshape(bb, NUM_PARTS, 128, nn)
    pred = jnp.einsum('pgc,bpcn->bpgn', ppg["W"], h2) + ppg["b"][None, :, :, None]
    pred = pred.reshape(bb, NUM_PARTS * NUM_CLASSES, nn)
    weighted = (pred.reshape(bb, NUM_CLASSES, NUM_PARTS, nn) * parts_sm[:, None, :, :]).sum(axis=2)
    out = jnp.concatenate([parts, weighted], axis=1)
    return _pallas_identity(out)
